# baseline (speedup 1.0000x reference)
import functools

import jax
import jax.numpy as jnp
from jax import lax
from jax.experimental import pallas as pl
from jax.experimental.pallas import tpu as pltpu
from jax.experimental.pallas import tpu_sc as plsc

EMB = 128
N_NODES = 100000
N_UNARY = 200000
N_BINARY = 200000

NC = 2
NS = 16
NW = NC * NS

CH = 512
IR = CH // 128

U_CHUNKS = 416
B_CHUNKS = 800
U_PAD = U_CHUNKS * CH
B_PAD = B_CHUNKS * CH


def _mlp_resid_body(x_ref, w1_ref, b1_ref, w2_ref, b2_ref, o_ref):
    x = x_ref[...]
    h = jnp.maximum(
        jnp.dot(x, w1_ref[...], preferred_element_type=jnp.float32) + b1_ref[...],
        0.0,
    )
    o_ref[...] = x + jnp.dot(h, w2_ref[...], preferred_element_type=jnp.float32) + b2_ref[...]


def _mlp_resid_alias_body(f_ref, x_ref, w1_ref, b1_ref, w2_ref, b2_ref, o_ref):
    del f_ref
    _mlp_resid_body(x_ref, w1_ref, b1_ref, w2_ref, b2_ref, o_ref)


def _unary_precompute(x, w1, b1, w2, b2):
    blk = 2000
    grid = N_NODES // blk
    return pl.pallas_call(
        _mlp_resid_body,
        grid=(grid,),
        in_specs=[
            pl.BlockSpec((blk, EMB), lambda i: (i, 0)),
            pl.BlockSpec((EMB, EMB), lambda i: (0, 0)),
            pl.BlockSpec((1, EMB), lambda i: (0, 0)),
            pl.BlockSpec((EMB, EMB), lambda i: (0, 0)),
            pl.BlockSpec((1, EMB), lambda i: (0, 0)),
        ],
        out_specs=pl.BlockSpec((blk, EMB), lambda i: (i, 0)),
        out_shape=jax.ShapeDtypeStruct((N_NODES, EMB), jnp.float32),
    )(x, w1, b1.reshape(1, EMB), w2, b2.reshape(1, EMB))


def _binary_mlp_into(f2, eb2, w1, b1, w2, b2):
    blk = 800
    grid = N_BINARY // blk
    off = 100000 // blk
    return pl.pallas_call(
        _mlp_resid_alias_body,
        grid=(grid,),
        in_specs=[
            pl.BlockSpec(memory_space=pl.ANY),
            pl.BlockSpec((blk, 2 * EMB), lambda i: (i, 0)),
            pl.BlockSpec((2 * EMB, 2 * EMB), lambda i: (0, 0)),
            pl.BlockSpec((1, 2 * EMB), lambda i: (0, 0)),
            pl.BlockSpec((2 * EMB, 2 * EMB), lambda i: (0, 0)),
            pl.BlockSpec((1, 2 * EMB), lambda i: (0, 0)),
        ],
        out_specs=pl.BlockSpec((blk, 2 * EMB), lambda i, off=off: (i + off, 0)),
        out_shape=jax.ShapeDtypeStruct((300000, 2 * EMB), jnp.float32),
        input_output_aliases={0: 0},
    )(f2, eb2, w1, b1.reshape(1, 2 * EMB), w2, b2.reshape(1, 2 * EMB))


def _sc_gather(x, u, idx_u, idx_b):
    mesh = plsc.VectorSubcoreMesh(core_axis_name="c", subcore_axis_name="s")

    @functools.partial(
        pl.kernel,
        mesh=mesh,
        out_type=[
            jax.ShapeDtypeStruct((3 * N_UNARY, EMB), jnp.float32),
            jax.ShapeDtypeStruct((B_PAD, EMB), jnp.float32),
        ],
        scratch_types=[
            pltpu.VMEM((IR, 128), jnp.int32),
            pltpu.VMEM((CH, EMB), jnp.float32),
            pltpu.SemaphoreType.DMA,
        ],
    )
    def k(x_hbm, u_hbm, idxu_hbm, idxb_hbm, f_hbm, eb_hbm, idx_v, rows_v, sem):
        wid = lax.axis_index("s") * NC + lax.axis_index("c")

        def region(table, idx_hbm, out_hbm, n_chunks):
            def body(t, carry):
                chunk = wid + NW * t
                pltpu.sync_copy(idx_hbm.at[pl.ds(chunk * IR, IR)], idx_v)
                cops = [
                    pltpu.async_copy(
                        table.at[idx_v.at[j]],
                        rows_v.at[pl.ds(j * 128, 128)],
                        sem,
                    )
                    for j in range(IR)
                ]
                for cop in cops:
                    cop.wait()
                pltpu.sync_copy(rows_v, out_hbm.at[pl.ds(chunk * CH, CH)])
                return carry

            lax.fori_loop(0, n_chunks // NW, body, 0)

        region(u_hbm, idxu_hbm, f_hbm, U_CHUNKS)
        region(x_hbm, idxb_hbm, eb_hbm, B_CHUNKS)

    return k(x, u, idx_u, idx_b)


def kernel(node_embeddings, rel_unary_indices, rel_binary_indices,
           u_W1, u_b1, u_W2, u_b2, b_W1, b_b1, b_W2, b_b2):
    x = node_embeddings
    u = _unary_precompute(x, u_W1, u_b1, u_W2, u_b2)

    idx_u = jnp.concatenate(
        [rel_unary_indices.astype(jnp.int32),
         jnp.zeros((U_PAD - N_UNARY,), jnp.int32)]
    ).reshape(-1, 128)
    idx_b = jnp.concatenate(
        [rel_binary_indices.astype(jnp.int32),
         jnp.zeros((B_PAD - 2 * N_BINARY,), jnp.int32)]
    ).reshape(-1, 128)

    f, eb = _sc_gather(x, u, idx_u, idx_b)

    f2 = f.reshape(300000, 2 * EMB)
    eb2 = eb.reshape(B_PAD // 2, 2 * EMB)
    out = _binary_mlp_into(f2, eb2, b_W1, b_b1, b_W2, b_b2)

    output_messages = out.reshape(3 * N_UNARY, EMB)
    output_indices = jnp.concatenate([rel_unary_indices, rel_binary_indices])
    return output_messages, output_indices


# speedup vs baseline: 1.4809x; 1.4809x over previous
import functools

import jax
import jax.numpy as jnp
from jax import lax
from jax.experimental import pallas as pl
from jax.experimental.pallas import tpu as pltpu
from jax.experimental.pallas import tpu_sc as plsc

EMB = 128
N_NODES = 100000
N_UNARY = 200000
N_BINARY = 200000
N_OUT = N_UNARY + 2 * N_BINARY

NC = 2
NS = 16
NW = NC * NS

CH = 256
IR = CH // 128

KU = 26
KB = 50
U_PAD = KU * NW * CH
B_PAD = KB * NW * CH


def _unary_body(x_ref, w1_ref, b1_ref, w2_ref, b2_ref, o_ref):
    x = x_ref[...]
    h = jnp.maximum(
        jnp.dot(x, w1_ref[...], preferred_element_type=jnp.float32) + b1_ref[...],
        0.0,
    )
    o_ref[...] = x + jnp.dot(h, w2_ref[...], preferred_element_type=jnp.float32) + b2_ref[...]


def _binary_body(f_ref, x_ref, w1_ref, b1_ref, w2_ref, b2_ref, o_ref):
    del f_ref
    x = x_ref[...].reshape(-1, 2 * EMB)
    h = jnp.maximum(
        jnp.dot(x, w1_ref[...], preferred_element_type=jnp.float32) + b1_ref[...],
        0.0,
    )
    y = x + jnp.dot(h, w2_ref[...], preferred_element_type=jnp.float32) + b2_ref[...]
    o_ref[...] = y.reshape(-1, EMB)


def _unary_precompute(x, w1, b1, w2, b2):
    blk = 2000
    return pl.pallas_call(
        _unary_body,
        grid=(N_NODES // blk,),
        in_specs=[
            pl.BlockSpec((blk, EMB), lambda i: (i, 0)),
            pl.BlockSpec((EMB, EMB), lambda i: (0, 0)),
            pl.BlockSpec((1, EMB), lambda i: (0, 0)),
            pl.BlockSpec((EMB, EMB), lambda i: (0, 0)),
            pl.BlockSpec((1, EMB), lambda i: (0, 0)),
        ],
        out_specs=pl.BlockSpec((blk, EMB), lambda i: (i, 0)),
        out_shape=jax.ShapeDtypeStruct((N_NODES, EMB), jnp.float32),
    )(x, w1, b1.reshape(1, EMB), w2, b2.reshape(1, EMB))


def _binary_mlp_into(f, eb, w1, b1, w2, b2):
    blk = 1600
    grid = 2 * N_BINARY // blk
    off = N_UNARY // blk
    return pl.pallas_call(
        _binary_body,
        grid=(grid,),
        in_specs=[
            pl.BlockSpec(memory_space=pl.ANY),
            pl.BlockSpec((blk, EMB), lambda i: (i, 0)),
            pl.BlockSpec((2 * EMB, 2 * EMB), lambda i: (0, 0)),
            pl.BlockSpec((1, 2 * EMB), lambda i: (0, 0)),
            pl.BlockSpec((2 * EMB, 2 * EMB), lambda i: (0, 0)),
            pl.BlockSpec((1, 2 * EMB), lambda i: (0, 0)),
        ],
        out_specs=pl.BlockSpec((blk, EMB), lambda i, off=off: (i + off, 0)),
        out_shape=jax.ShapeDtypeStruct((N_OUT, EMB), jnp.float32),
        input_output_aliases={0: 0},
    )(f, eb, w1, b1.reshape(1, 2 * EMB), w2, b2.reshape(1, 2 * EMB))


def _sc_gather(x, u, idx_u, idx_b):
    mesh = plsc.VectorSubcoreMesh(core_axis_name="c", subcore_axis_name="s")

    @functools.partial(
        pl.kernel,
        mesh=mesh,
        out_type=[
            jax.ShapeDtypeStruct((N_OUT, EMB), jnp.float32),
            jax.ShapeDtypeStruct((B_PAD, EMB), jnp.float32),
        ],
        scratch_types=[
            pltpu.VMEM((IR, 128), jnp.int32),
            pltpu.VMEM((IR, 128), jnp.int32),
            pltpu.VMEM((CH, EMB), jnp.float32),
            pltpu.VMEM((CH, EMB), jnp.float32),
            pltpu.SemaphoreType.DMA,
            pltpu.SemaphoreType.DMA,
            pltpu.SemaphoreType.DMA,
            pltpu.SemaphoreType.DMA,
            pltpu.SemaphoreType.DMA,
            pltpu.SemaphoreType.DMA,
        ],
    )
    def k(x_hbm, u_hbm, idxu_hbm, idxb_hbm, f_hbm, eb_hbm,
          idx_v0, idx_v1, rows_v0, rows_v1,
          sem_i0, sem_i1, sem_g0, sem_g1, sem_o0, sem_o1):
        wid = lax.axis_index("s") * NC + lax.axis_index("c")
        bufs = (
            (idx_v0, rows_v0, sem_i0, sem_g0, sem_o0, idx_v1, sem_i1),
            (idx_v1, rows_v1, sem_i1, sem_g1, sem_o1, idx_v0, sem_i0),
        )

        def region(table, idx_hbm, out_hbm, k_per_worker):
            pltpu.async_copy(idx_hbm.at[pl.ds(wid * IR, IR)], idx_v0, sem_i0)

            def body(kk, carry):
                for b, (idxv, rowsv, semi, semg, semo, idxv_n, semi_n) in enumerate(bufs):
                    t = 2 * kk + b

                    @pl.when(t >= 2)
                    def _():
                        pltpu.make_async_copy(
                            out_hbm.at[pl.ds(0, CH)], rowsv, semo
                        ).wait()

                    pltpu.make_async_copy(
                        idx_hbm.at[pl.ds(0, IR)], idxv, semi
                    ).wait()
                    cops = [
                        pltpu.async_copy(
                            table.at[idxv.at[j]],
                            rowsv.at[pl.ds(j * 128, 128)],
                            semg,
                        )
                        for j in range(IR)
                    ]

                    @pl.when(t + 1 < k_per_worker)
                    def _():
                        nxt = (wid + NW * (t + 1)) * IR
                        pltpu.async_copy(
                            idx_hbm.at[pl.ds(nxt, IR)], idxv_n, semi_n
                        )

                    for cop in cops:
                        cop.wait()
                    pltpu.async_copy(
                        rowsv, out_hbm.at[pl.ds((wid + NW * t) * CH, CH)], semo
                    )
                return carry

            lax.fori_loop(0, k_per_worker // 2, body, 0)
            pltpu.make_async_copy(out_hbm.at[pl.ds(0, CH)], rows_v0, sem_o0).wait()
            pltpu.make_async_copy(out_hbm.at[pl.ds(0, CH)], rows_v1, sem_o1).wait()

        region(u_hbm, idxu_hbm, f_hbm, KU)
        region(x_hbm, idxb_hbm, eb_hbm, KB)

    return k(x, u, idx_u, idx_b)


def kernel(node_embeddings, rel_unary_indices, rel_binary_indices,
           u_W1, u_b1, u_W2, u_b2, b_W1, b_b1, b_W2, b_b2):
    x = node_embeddings
    u = _unary_precompute(x, u_W1, u_b1, u_W2, u_b2)

    idx_u = jnp.concatenate(
        [rel_unary_indices.astype(jnp.int32),
         jnp.zeros((U_PAD - N_UNARY,), jnp.int32)]
    ).reshape(-1, 128)
    idx_b = jnp.concatenate(
        [rel_binary_indices.astype(jnp.int32),
         jnp.zeros((B_PAD - 2 * N_BINARY,), jnp.int32)]
    ).reshape(-1, 128)

    f, eb = _sc_gather(x, u, idx_u, idx_b)
    out = _binary_mlp_into(f, eb, b_W1, b_b1, b_W2, b_b2)

    output_indices = jnp.concatenate([rel_unary_indices, rel_binary_indices])
    return out, output_indices
